# baseline (speedup 1.0000x reference)
import functools

import jax
import jax.numpy as jnp
from jax.experimental import pallas as pl
from jax.experimental.pallas import tpu as pltpu

REHEARSAL_WEIGHT = 0.5


def _tree_sum_rows(chunk, rows, E):
    z = chunk.reshape(rows // 8, 8, E)
    vals = [z[j] for j in range(rows // 8)]
    while len(vals) > 1:
        nxt = [a + b for a, b in zip(vals[0::2], vals[1::2])]
        if len(vals) % 2:
            nxt[-1] = nxt[-1] + vals[-1]
        vals = nxt
    return vals[0]


def _gating_loss_kernel(x_hbm, hc_ref, smk_ref, out_ref, buf, sems, *,
                        n_chunks, chunk_rows, n_rows, k):
    E = x_hbm.shape[2]
    T = x_hbm.shape[1]
    chunks_per_batch = T // chunk_rows

    copies = []
    for c in range(n_chunks):
        b = c // chunks_per_batch
        t = c % chunks_per_batch
        cp = pltpu.make_async_copy(
            x_hbm.at[b, pl.ds(t * chunk_rows, chunk_rows), :],
            buf.at[c],
            sems.at[c],
        )
        cp.start()
        copies.append(cp)

    acc = None
    for c in range(n_chunks):
        copies[c].wait()
        p = _tree_sum_rows(buf[c], chunk_rows, E)
        acc = p if acc is None else acc + p

    colsum = jnp.sum(acc, axis=0, keepdims=True)
    hc = hc_ref[...].reshape(1, E)

    expert_ids = jax.lax.broadcasted_iota(jnp.int32, (1, E), 1)
    sel = (expert_ids == smk_ref[0]).astype(jnp.float32)
    for j in range(1, k):
        sel = jnp.maximum(sel, (expert_ids == smk_ref[j]).astype(jnp.float32))

    inv_n = 1.0 / n_rows
    log_inv_k = -jnp.log(float(k))
    scount = jnp.sum(sel)
    ssum = jnp.sum(sel * colsum)
    smk_loss = scount * (1.0 / k) * log_inv_k - (1.0 / k) * ssum * inv_n

    clamped = jnp.clip(hc, -10.0, 10.0)
    m = jnp.max(clamped)
    lse = m + jnp.log(jnp.sum(jnp.exp(clamped - m)))
    r = clamped - lse
    p_r = jnp.exp(r)
    rehearsal_loss = jnp.sum(p_r * r) - jnp.sum(p_r * colsum) * inv_n

    use_rehearsal = jnp.sum(jnp.abs(hc)) > 0.0
    loss = jnp.where(
        use_rehearsal,
        (1.0 - REHEARSAL_WEIGHT) * smk_loss + REHEARSAL_WEIGHT * rehearsal_loss,
        smk_loss,
    )
    out_ref[0] = loss


def kernel(log_probs, history_context, smk_indices):
    B, T, E = log_probs.shape
    n_rows = B * T
    k = smk_indices.shape[0]

    chunks_per_batch = 4
    n_chunks = B * chunks_per_batch
    chunk_rows = T // chunks_per_batch

    out = pl.pallas_call(
        functools.partial(_gating_loss_kernel, n_chunks=n_chunks,
                          chunk_rows=chunk_rows, n_rows=n_rows, k=k),
        in_specs=[
            pl.BlockSpec(memory_space=pl.ANY),
            pl.BlockSpec(memory_space=pltpu.VMEM),
            pl.BlockSpec(memory_space=pltpu.SMEM),
        ],
        out_specs=pl.BlockSpec(memory_space=pltpu.SMEM),
        out_shape=jax.ShapeDtypeStruct((1,), jnp.float32),
        scratch_shapes=[
            pltpu.VMEM((n_chunks, chunk_rows, E), jnp.float32),
            pltpu.SemaphoreType.DMA((n_chunks,)),
        ],
    )(log_probs, history_context, smk_indices)
    return out[0]
